# baseline (speedup 1.0000x reference)
import functools

import jax
import jax.numpy as jnp
from jax import lax
from jax.experimental import pallas as pl
from jax.experimental.pallas import tpu as pltpu
from jax.experimental.pallas import tpu_sc as plsc

L = 16
NC, NS = 1, 16
NW = NC * NS
B = 16384
BPW = B // NW
V = 100
VPAD = 112
NT = VPAD // L
SG = VPAD + L


def _body(stage_hbm, idx_hbm, out_hbm, stage_v, idx_v, out_v, sem1, sem2,
          sem3):
    wid = lax.axis_index("s") * NC + lax.axis_index("c")
    base = wid * BPW

    half = BPW // 2
    cp_i0 = pltpu.async_copy(idx_hbm.at[pl.ds(base, half)],
                             idx_v.at[pl.ds(0, half)], sem1)
    cp_stage = pltpu.async_copy(stage_hbm, stage_v, sem2)
    cp_i1 = pltpu.async_copy(idx_hbm.at[pl.ds(base + half, half)],
                             idx_v.at[pl.ds(half, half)], sem3)
    cp_stage.wait()

    wv = stage_v[pl.ds(VPAD, L)]
    w1 = jnp.full((L,), wv[0], jnp.float32)
    b1 = jnp.full((L,), wv[1], jnp.float32)
    w2 = jnp.full((L,), wv[2], jnp.float32)
    b2 = jnp.full((L,), wv[3], jnp.float32)

    ftab = []
    for t in range(NT):
        x = stage_v[pl.ds(t * L, L)]
        ftab.append(w2 * jnp.maximum(w1 * x + b1, 0.0) + b2)

    UNROLL = 4

    def step(i, carry):
        for u in range(UNROLL):
            off = i * (UNROLL * L) + u * L
            idx = idx_v[pl.ds(off, L)]
            lane = lax.bitwise_and(idx, L - 1)
            grp = lax.shift_right_logical(idx, 4)
            acc = ftab[0].at[lane].get(mode="promise_in_bounds")
            for t in range(1, NT):
                g = ftab[t].at[lane].get(mode="promise_in_bounds")
                acc = jnp.where(grp == t, g, acc)
            out_v[pl.ds(off, L)] = acc
        return carry

    nsteps = half // (UNROLL * L)
    cp_i0.wait()
    lax.fori_loop(0, nsteps, step, 0, unroll=False)
    cp_i1.wait()
    lax.fori_loop(nsteps, 2 * nsteps, step, 0, unroll=False)

    pltpu.sync_copy(out_v, out_hbm.at[pl.ds(base, BPW)])


@jax.jit
def _run(stage, idx):
    mesh = plsc.VectorSubcoreMesh(core_axis_name="c", subcore_axis_name="s",
                                  num_cores=NC)
    return pl.kernel(
        _body,
        out_type=jax.ShapeDtypeStruct((B,), jnp.float32),
        mesh=mesh,
        scratch_types=[
            pltpu.VMEM((SG,), jnp.float32),
            pltpu.VMEM((BPW,), jnp.int32),
            pltpu.VMEM((BPW,), jnp.float32),
            pltpu.SemaphoreType.DMA,
            pltpu.SemaphoreType.DMA,
            pltpu.SemaphoreType.DMA,
        ],
    )(stage, idx)


def kernel(t_list, t_hscore, W1, b1, W2, b2):
    stage = jnp.concatenate([
        t_hscore.astype(jnp.float32),
        jnp.zeros((VPAD - V,), jnp.float32),
        W1.reshape((1,)).astype(jnp.float32),
        b1.reshape((1,)).astype(jnp.float32),
        W2.reshape((1,)).astype(jnp.float32),
        b2.reshape((1,)).astype(jnp.float32),
        jnp.zeros((L - 4,), jnp.float32),
    ])
    return _run(stage, t_list.astype(jnp.int32))


# speedup vs baseline: 1.0028x; 1.0028x over previous
import functools

import jax
import jax.numpy as jnp
from jax import lax
from jax.experimental import pallas as pl
from jax.experimental.pallas import tpu as pltpu
from jax.experimental.pallas import tpu_sc as plsc

L = 16
NC, NS = 1, 16
NW = NC * NS
B = 16384
BPW = B // NW
V = 100
VPAD = 112
NT = VPAD // L
SG = VPAD + L


def _body(stage_hbm, idx_hbm, out_hbm, stage_v, idx_v, out_v, sem1, sem2):
    wid = lax.axis_index("s") * NC + lax.axis_index("c")
    base = wid * BPW

    cp_idx = pltpu.async_copy(idx_hbm.at[pl.ds(base, BPW)], idx_v, sem1)
    cp_stage = pltpu.async_copy(stage_hbm, stage_v, sem2)
    cp_stage.wait()

    wv = stage_v[pl.ds(VPAD, L)]
    w1 = jnp.full((L,), wv[0], jnp.float32)
    b1 = jnp.full((L,), wv[1], jnp.float32)
    w2 = jnp.full((L,), wv[2], jnp.float32)
    b2 = jnp.full((L,), wv[3], jnp.float32)

    ftab = []
    for t in range(NT):
        x = stage_v[pl.ds(t * L, L)]
        ftab.append(w2 * jnp.maximum(w1 * x + b1, 0.0) + b2)

    cp_idx.wait()

    UNROLL = 2

    def step(i, carry):
        for u in range(UNROLL):
            off = i * (UNROLL * L) + u * L
            idx = idx_v[pl.ds(off, L)]
            lane = lax.bitwise_and(idx, L - 1)
            grp = lax.shift_right_logical(idx, 4)
            acc = ftab[0].at[lane].get(mode="promise_in_bounds")
            for t in range(1, NT):
                g = ftab[t].at[lane].get(mode="promise_in_bounds")
                acc = jnp.where(grp == t, g, acc)
            out_v[pl.ds(off, L)] = acc
        return carry

    lax.fori_loop(0, BPW // (UNROLL * L), step, 0, unroll=False)

    pltpu.sync_copy(out_v, out_hbm.at[pl.ds(base, BPW)])


@jax.jit
def _run(stage, idx):
    mesh = plsc.VectorSubcoreMesh(core_axis_name="c", subcore_axis_name="s",
                                  num_cores=NC)
    return pl.kernel(
        _body,
        out_type=jax.ShapeDtypeStruct((B,), jnp.float32),
        mesh=mesh,
        scratch_types=[
            pltpu.VMEM((SG,), jnp.float32),
            pltpu.VMEM((BPW,), jnp.int32),
            pltpu.VMEM((BPW,), jnp.float32),
            pltpu.SemaphoreType.DMA,
            pltpu.SemaphoreType.DMA,
        ],
    )(stage, idx)


def kernel(t_list, t_hscore, W1, b1, W2, b2):
    stage = jnp.concatenate([
        t_hscore.astype(jnp.float32),
        jnp.zeros((VPAD - V,), jnp.float32),
        W1.reshape((1,)).astype(jnp.float32),
        b1.reshape((1,)).astype(jnp.float32),
        W2.reshape((1,)).astype(jnp.float32),
        b2.reshape((1,)).astype(jnp.float32),
        jnp.zeros((L - 4,), jnp.float32),
    ])
    return _run(stage, t_list.astype(jnp.int32))


# speedup vs baseline: 1.0091x; 1.0063x over previous
import functools

import jax
import jax.numpy as jnp
from jax import lax
from jax.experimental import pallas as pl
from jax.experimental.pallas import tpu as pltpu
from jax.experimental.pallas import tpu_sc as plsc

L = 16
NC, NS = 1, 16
NW = NC * NS
B = 16384
BPW = B // NW
V = 100
VPAD = 112
NT = VPAD // L
SG = VPAD + L


def _body(stage_hbm, idx_hbm, out_hbm, stage_v, idx_v, out_v, sem1, sem2):
    wid = lax.axis_index("s") * NC + lax.axis_index("c")
    base = wid * BPW

    cp_idx = pltpu.async_copy(idx_hbm.at[pl.ds(base, BPW)], idx_v, sem1)
    cp_stage = pltpu.async_copy(stage_hbm, stage_v, sem2)
    cp_stage.wait()

    wv = stage_v[pl.ds(VPAD, L)]
    w1 = jnp.full((L,), wv[0], jnp.float32)
    b1 = jnp.full((L,), wv[1], jnp.float32)
    w2 = jnp.full((L,), wv[2], jnp.float32)
    b2 = jnp.full((L,), wv[3], jnp.float32)

    ftab = []
    for t in range(NT):
        x = stage_v[pl.ds(t * L, L)]
        ftab.append(w2 * jnp.maximum(w1 * x + b1, 0.0) + b2)

    cp_idx.wait()

    UNROLL = 4

    def step(i, carry):
        for u in range(UNROLL):
            off = i * (UNROLL * L) + u * L
            idx = idx_v[pl.ds(off, L)]
            lane = lax.bitwise_and(idx, L - 1)
            grp = lax.shift_right_logical(idx, 4)
            acc = ftab[0].at[lane].get(mode="promise_in_bounds")
            for t in range(1, NT):
                g = ftab[t].at[lane].get(mode="promise_in_bounds")
                acc = jnp.where(grp == t, g, acc)
            out_v[pl.ds(off, L)] = acc
        return carry

    lax.fori_loop(0, BPW // (UNROLL * L), step, 0, unroll=False)

    pltpu.sync_copy(out_v, out_hbm.at[pl.ds(base, BPW)])


@jax.jit
def _run(stage, idx):
    mesh = plsc.VectorSubcoreMesh(core_axis_name="c", subcore_axis_name="s",
                                  num_cores=NC)
    return pl.kernel(
        _body,
        out_type=jax.ShapeDtypeStruct((B,), jnp.float32),
        mesh=mesh,
        scratch_types=[
            pltpu.VMEM((SG,), jnp.float32),
            pltpu.VMEM((BPW,), jnp.int32),
            pltpu.VMEM((BPW,), jnp.float32),
            pltpu.SemaphoreType.DMA,
            pltpu.SemaphoreType.DMA,
        ],
    )(stage, idx)


def kernel(t_list, t_hscore, W1, b1, W2, b2):
    stage = jnp.concatenate([
        t_hscore.astype(jnp.float32),
        jnp.zeros((VPAD - V,), jnp.float32),
        W1.reshape((1,)).astype(jnp.float32),
        b1.reshape((1,)).astype(jnp.float32),
        W2.reshape((1,)).astype(jnp.float32),
        b2.reshape((1,)).astype(jnp.float32),
        jnp.zeros((L - 4,), jnp.float32),
    ])
    return _run(stage, t_list.astype(jnp.int32))
